# Initial kernel scaffold; baseline (speedup 1.0000x reference)
import jax
import jax.numpy as jnp
from jax.experimental import pallas as pl


def kernel(preds, targets):
    raise NotImplementedError("write your pallas kernel here")


# speedup vs baseline: 1541.2458x; 1541.2458x over previous
import functools

import jax
import jax.numpy as jnp
from jax import lax
from jax.experimental import pallas as pl
from jax.experimental.pallas import tpu as pltpu
from jax.experimental.pallas import tpu_sc as plsc

EPS = 1e-07
NT = 200
N_TOTAL = 4194304
NC, NS, LANES = 2, 16, 16
NW = NC * NS
EPT = N_TOTAL // NW
CH = 16384
NCHUNK = EPT // CH
BINS = 512
ACCN = LANES * BINS


def _thr_table():
    thr = [(i + 1) * 1.0 / (NT - 1) for i in range(NT - 2)]
    thr = [0.0 - EPS] + thr + [1.0 + EPS] + [2.0] * 8
    return jnp.asarray(thr, dtype=jnp.float32)


def _sc_hist(preds, targets, thr):
    mesh = plsc.VectorSubcoreMesh(core_axis_name="c", subcore_axis_name="s")

    @functools.partial(
        pl.kernel,
        out_type=jax.ShapeDtypeStruct((NW, ACCN), jnp.float32),
        mesh=mesh,
        compiler_params=pltpu.CompilerParams(needs_layout_passes=False),
        scratch_types=[
            pltpu.VMEM((CH,), jnp.float32),
            pltpu.VMEM((CH,), jnp.float32),
            pltpu.VMEM((208,), jnp.float32),
            pltpu.VMEM((ACCN,), jnp.float32),
        ],
    )
    def k(preds_hbm, targets_hbm, thr_hbm, out_hbm, pbuf, tbuf, thrv, acc):
        wid = lax.axis_index("s") * NC + lax.axis_index("c")
        base = wid * EPT
        pltpu.sync_copy(thr_hbm, thrv)

        zeros16 = jnp.zeros((LANES,), jnp.float32)

        def zbody(i, carry):
            acc[pl.ds(i * LANES, LANES)] = zeros16
            return carry

        lax.fori_loop(0, ACCN // LANES, zbody, 0)

        ones16 = jnp.full((LANES,), 1.0, jnp.float32)
        lane_off = lax.iota(jnp.int32, 16) * BINS

        for c in range(NCHUNK):
            off = base + c * CH
            pltpu.sync_copy(preds_hbm.at[pl.ds(off, CH)], pbuf)
            pltpu.sync_copy(targets_hbm.at[pl.ds(off, CH)], tbuf)

            def body(i, carry):
                s = pl.ds(i * LANES, LANES)
                p = pbuf[s]
                g = (p * 199.0 + 0.5).astype(jnp.int32)
                g = jnp.minimum(jnp.maximum(g, 0), 199)
                tv = plsc.load_gather(thrv, [g])
                b = g + (tv < p).astype(jnp.int32)
                t = tbuf[s]
                idx = b + (t == 1.0).astype(jnp.int32) * 256 + lane_off
                plsc.addupdate_scatter(acc, [idx], ones16)
                return carry

            lax.fori_loop(0, CH // LANES, body, 0)

        pltpu.sync_copy(acc, out_hbm.at[wid])

    return k(preds, targets, thr)


def _tc_auc(partials):
    def body(h_ref, o_ref):
        h = h_ref[...]
        s = jnp.sum(h, axis=0, keepdims=True)
        h_nt = s[:, 0:256]
        h_t = s[:, 256:512]
        r = lax.broadcasted_iota(jnp.int32, (256, 256), 0)
        cc = lax.broadcasted_iota(jnp.int32, (256, 256), 1)
        m_cum = (r <= cc).astype(jnp.float32)
        m_rev = (jnp.maximum(r - 1, 0) >= cc).astype(jnp.float32)
        dot = functools.partial(lax.dot, precision=lax.Precision.HIGHEST)
        tn = dot(h_nt, m_cum)
        fn = dot(h_t, m_cum)
        fp = dot(h_nt, m_rev)
        tp = dot(h_t, m_rev)
        x = fp / (fp + tn + EPS)
        y = (tp + EPS) / (tp + fn + EPS)
        shift = (r == cc + 1).astype(jnp.float32)
        xs = dot(x, shift)
        ys = dot(y, shift)
        j = lax.broadcasted_iota(jnp.int32, (1, 256), 1)
        terms = jnp.where(j <= NT - 2, (x - xs) * (y + ys) * 0.5, 0.0)
        o_ref[...] = jnp.sum(terms, axis=1, keepdims=True)

    return pl.pallas_call(
        body,
        out_shape=jax.ShapeDtypeStruct((1, 1), jnp.float32),
    )(partials)


def kernel(preds, targets):
    p = preds.reshape(-1)
    t = targets.reshape(-1)
    hist = _sc_hist(p, t, _thr_table())
    roc = _tc_auc(hist.reshape(NW * LANES, BINS))
    return roc.reshape(())


# speedup vs baseline: 5829.9057x; 3.7826x over previous
import functools

import jax
import jax.numpy as jnp
from jax import lax
from jax.experimental import pallas as pl
from jax.experimental.pallas import tpu as pltpu
from jax.experimental.pallas import tpu_sc as plsc

EPS = 1e-07
NT = 200
N_TOTAL = 4194304
NC, NS, LANES = 2, 16, 16
NW = NC * NS
EPT = N_TOTAL // NW
CH = 16384
NCHUNK = EPT // CH
BINS = 512
ACCN = LANES * BINS


def _thr_table():
    thr = [(i + 1) * 1.0 / (NT - 1) for i in range(NT - 2)]
    thr = [0.0 - EPS] + thr + [1.0 + EPS] + [2.0] * 8
    return jnp.asarray(thr, dtype=jnp.float32)


def _sc_hist(preds, targets, thr):
    mesh = plsc.VectorSubcoreMesh(core_axis_name="c", subcore_axis_name="s")

    @functools.partial(
        pl.kernel,
        out_type=jax.ShapeDtypeStruct((NW, ACCN), jnp.float32),
        mesh=mesh,
        compiler_params=pltpu.CompilerParams(needs_layout_passes=False),
        scratch_types=[
            pltpu.VMEM((CH,), jnp.float32),
            pltpu.VMEM((CH,), jnp.float32),
            pltpu.VMEM((CH,), jnp.float32),
            pltpu.VMEM((CH,), jnp.float32),
            pltpu.VMEM((208,), jnp.float32),
            pltpu.VMEM((ACCN,), jnp.float32),
            pltpu.SemaphoreType.DMA,
            pltpu.SemaphoreType.DMA,
            pltpu.SemaphoreType.DMA,
            pltpu.SemaphoreType.DMA,
        ],
    )
    def k(preds_hbm, targets_hbm, thr_hbm, out_hbm,
          pbuf0, tbuf0, pbuf1, tbuf1, thrv, acc, sp0, st0, sp1, st1):
        wid = lax.axis_index("s") * NC + lax.axis_index("c")
        base = wid * EPT
        bufs = ((pbuf0, tbuf0, sp0, st0), (pbuf1, tbuf1, sp1, st1))

        def start(c):
            pb, tb, ps, ts = bufs[c % 2]
            off = base + c * CH
            cp = pltpu.async_copy(preds_hbm.at[pl.ds(off, CH)], pb, ps)
            ct = pltpu.async_copy(targets_hbm.at[pl.ds(off, CH)], tb, ts)
            return cp, ct

        pending = start(0)
        pltpu.sync_copy(thr_hbm, thrv)

        zeros16 = jnp.zeros((LANES,), jnp.float32)

        @plsc.parallel_loop(0, ACCN, step=LANES, unroll=8)
        def _zero(i):
            acc[pl.ds(i, LANES)] = zeros16

        ones16 = jnp.full((LANES,), 1.0, jnp.float32)
        lane_off = lax.iota(jnp.int32, 16) * BINS

        for c in range(NCHUNK):
            cur = pending
            if c + 1 < NCHUNK:
                pending = start(c + 1)
            cur[0].wait()
            cur[1].wait()
            pb, tb = bufs[c % 2][0], bufs[c % 2][1]

            @plsc.parallel_loop(0, CH, step=LANES, unroll=8)
            def _body(i):
                s = pl.ds(i, LANES)
                p = pb[s]
                g = (p * 199.0 + 0.5).astype(jnp.int32)
                g = jnp.minimum(jnp.maximum(g, 0), 199)
                tv = plsc.load_gather(thrv, [g])
                b = g + (tv < p).astype(jnp.int32)
                t = tb[s]
                idx = b + (t == 1.0).astype(jnp.int32) * 256 + lane_off
                plsc.addupdate_scatter(acc, [idx], ones16)

        pltpu.sync_copy(acc, out_hbm.at[wid])

    return k(preds, targets, thr)


def _tc_auc(partials):
    def body(h_ref, o_ref):
        h = h_ref[...]
        s = jnp.sum(h, axis=0, keepdims=True)
        h_nt = s[:, 0:256]
        h_t = s[:, 256:512]
        r = lax.broadcasted_iota(jnp.int32, (256, 256), 0)
        cc = lax.broadcasted_iota(jnp.int32, (256, 256), 1)
        m_cum = (r <= cc).astype(jnp.float32)
        m_rev = (jnp.maximum(r - 1, 0) >= cc).astype(jnp.float32)
        dot = functools.partial(lax.dot, precision=lax.Precision.HIGHEST)
        tn = dot(h_nt, m_cum)
        fn = dot(h_t, m_cum)
        fp = dot(h_nt, m_rev)
        tp = dot(h_t, m_rev)
        x = fp / (fp + tn + EPS)
        y = (tp + EPS) / (tp + fn + EPS)
        shift = (r == cc + 1).astype(jnp.float32)
        xs = dot(x, shift)
        ys = dot(y, shift)
        j = lax.broadcasted_iota(jnp.int32, (1, 256), 1)
        terms = jnp.where(j <= NT - 2, (x - xs) * (y + ys) * 0.5, 0.0)
        o_ref[...] = jnp.sum(terms, axis=1, keepdims=True)

    return pl.pallas_call(
        body,
        out_shape=jax.ShapeDtypeStruct((1, 1), jnp.float32),
    )(partials)


def kernel(preds, targets):
    p = preds.reshape(-1)
    t = targets.reshape(-1)
    hist = _sc_hist(p, t, _thr_table())
    roc = _tc_auc(hist.reshape(NW * LANES, BINS))
    return roc.reshape(())


# speedup vs baseline: 6331.0375x; 1.0860x over previous
import functools

import jax
import jax.numpy as jnp
from jax import lax
from jax.experimental import pallas as pl
from jax.experimental.pallas import tpu as pltpu
from jax.experimental.pallas import tpu_sc as plsc

EPS = 1e-07
NT = 200
N_TOTAL = 4194304
NC, NS, LANES = 2, 16, 16
NW = NC * NS
EPT = N_TOTAL // NW
CH = 16384
NCHUNK = EPT // CH
BINS = 512
ACCN = LANES * BINS


def _thr_table():
    thr = [(i + 1) * 1.0 / (NT - 1) for i in range(NT - 2)]
    thr = [0.0 - EPS] + thr + [1.0 + EPS] + [2.0] * 8
    return jnp.asarray(thr, dtype=jnp.float32)


def _sc_hist(preds, targets, thr):
    mesh = plsc.VectorSubcoreMesh(core_axis_name="c", subcore_axis_name="s")

    @functools.partial(
        pl.kernel,
        out_type=jax.ShapeDtypeStruct((NW, ACCN), jnp.float32),
        mesh=mesh,
        compiler_params=pltpu.CompilerParams(needs_layout_passes=False),
        scratch_types=[
            pltpu.VMEM((CH,), jnp.float32),
            pltpu.VMEM((CH,), jnp.float32),
            pltpu.VMEM((CH,), jnp.float32),
            pltpu.VMEM((CH,), jnp.float32),
            pltpu.VMEM((208,), jnp.float32),
            pltpu.VMEM((ACCN,), jnp.float32),
            pltpu.SemaphoreType.DMA,
            pltpu.SemaphoreType.DMA,
            pltpu.SemaphoreType.DMA,
            pltpu.SemaphoreType.DMA,
        ],
    )
    def k(preds_hbm, targets_hbm, thr_hbm, out_hbm,
          pbuf0, tbuf0, pbuf1, tbuf1, thrv, acc, sp0, st0, sp1, st1):
        wid = lax.axis_index("s") * NC + lax.axis_index("c")
        base = wid * EPT
        bufs = ((pbuf0, tbuf0, sp0, st0), (pbuf1, tbuf1, sp1, st1))

        def start(c):
            pb, tb, ps, ts = bufs[c % 2]
            off = base + c * CH
            cp = pltpu.async_copy(preds_hbm.at[pl.ds(off, CH)], pb, ps)
            ct = pltpu.async_copy(targets_hbm.at[pl.ds(off, CH)], tb, ts)
            return cp, ct

        pending = start(0)
        pltpu.sync_copy(thr_hbm, thrv)

        zeros16 = jnp.zeros((LANES,), jnp.float32)

        @plsc.parallel_loop(0, ACCN, step=LANES, unroll=8)
        def _zero(i):
            acc[pl.ds(i, LANES)] = zeros16

        ones16 = jnp.full((LANES,), 1.0, jnp.float32)
        lane_off = lax.iota(jnp.int32, 16) * BINS
        magic = jnp.float32(12582912.0)
        magic_bits = jnp.full((LANES,), 1262485504, jnp.int32)
        lane_base = lane_off - magic_bits

        for c in range(NCHUNK):
            cur = pending
            if c + 1 < NCHUNK:
                pending = start(c + 1)
            cur[0].wait()
            cur[1].wait()
            pb, tb = bufs[c % 2][0], bufs[c % 2][1]

            @plsc.parallel_loop(0, CH, step=LANES, unroll=16)
            def _body(i):
                s = pl.ds(i, LANES)
                p = pb[s]
                v = p * 199.0 + magic
                g = plsc.bitcast(v, jnp.int32) - magic_bits
                tv = plsc.load_gather(thrv, [g])
                t = tb[s]
                idx = (plsc.bitcast(v, jnp.int32) + lane_base
                       + jnp.where(tv < p, 1, 0)
                       + jnp.where(t == 1.0, 256, 0))
                plsc.addupdate_scatter(acc, [idx], ones16)

        pltpu.sync_copy(acc, out_hbm.at[wid])

    return k(preds, targets, thr)


def _tc_auc(partials):
    def body(h_ref, o_ref):
        h = h_ref[...]
        s = jnp.sum(h, axis=0, keepdims=True)
        h_nt = s[:, 0:256]
        h_t = s[:, 256:512]
        r = lax.broadcasted_iota(jnp.int32, (256, 256), 0)
        cc = lax.broadcasted_iota(jnp.int32, (256, 256), 1)
        m_cum = (r <= cc).astype(jnp.float32)
        m_rev = (jnp.maximum(r - 1, 0) >= cc).astype(jnp.float32)
        dot = functools.partial(lax.dot, precision=lax.Precision.HIGHEST)
        tn = dot(h_nt, m_cum)
        fn = dot(h_t, m_cum)
        fp = dot(h_nt, m_rev)
        tp = dot(h_t, m_rev)
        x = fp / (fp + tn + EPS)
        y = (tp + EPS) / (tp + fn + EPS)
        shift = (r == cc + 1).astype(jnp.float32)
        xs = dot(x, shift)
        ys = dot(y, shift)
        j = lax.broadcasted_iota(jnp.int32, (1, 256), 1)
        terms = jnp.where(j <= NT - 2, (x - xs) * (y + ys) * 0.5, 0.0)
        o_ref[...] = jnp.sum(terms, axis=1, keepdims=True)

    return pl.pallas_call(
        body,
        out_shape=jax.ShapeDtypeStruct((1, 1), jnp.float32),
    )(partials)


def kernel(preds, targets):
    p = preds.reshape(-1)
    t = targets.reshape(-1)
    hist = _sc_hist(p, t, _thr_table())
    roc = _tc_auc(hist.reshape(NW * LANES, BINS))
    return roc.reshape(())


# speedup vs baseline: 7060.2384x; 1.1152x over previous
import functools

import jax
import jax.numpy as jnp
from jax import lax
from jax.experimental import pallas as pl
from jax.experimental.pallas import tpu as pltpu
from jax.experimental.pallas import tpu_sc as plsc

EPS = 1e-07
NT = 200
N_TOTAL = 4194304
NC, NS, LANES = 2, 16, 16
NW = NC * NS
EPT = N_TOTAL // NW
CH = 16384
NCHUNK = EPT // CH
BINS = 512
ACCN = LANES * BINS


def _thr_table():
    thr = [(i + 1) * 1.0 / (NT - 1) for i in range(NT - 2)]
    thr = [0.0 - EPS] + thr + [1.0 + EPS] + [2.0] * 8
    return jnp.repeat(jnp.asarray(thr, dtype=jnp.float32), LANES)


def _sc_hist(preds, targets, thr):
    mesh = plsc.VectorSubcoreMesh(core_axis_name="c", subcore_axis_name="s")

    @functools.partial(
        pl.kernel,
        out_type=jax.ShapeDtypeStruct((NW, ACCN), jnp.float32),
        mesh=mesh,
        compiler_params=pltpu.CompilerParams(needs_layout_passes=False),
        scratch_types=[
            pltpu.VMEM((CH,), jnp.float32),
            pltpu.VMEM((CH,), jnp.float32),
            pltpu.VMEM((CH,), jnp.float32),
            pltpu.VMEM((CH,), jnp.float32),
            pltpu.VMEM((208 * LANES,), jnp.float32),
            pltpu.VMEM((ACCN,), jnp.float32),
            pltpu.SemaphoreType.DMA,
            pltpu.SemaphoreType.DMA,
            pltpu.SemaphoreType.DMA,
            pltpu.SemaphoreType.DMA,
        ],
    )
    def k(preds_hbm, targets_hbm, thr_hbm, out_hbm,
          pbuf0, tbuf0, pbuf1, tbuf1, thrv, acc, sp0, st0, sp1, st1):
        wid = lax.axis_index("s") * NC + lax.axis_index("c")
        base = wid * EPT
        bufs = ((pbuf0, tbuf0, sp0, st0), (pbuf1, tbuf1, sp1, st1))

        def start(c):
            pb, tb, ps, ts = bufs[c % 2]
            off = base + c * CH
            cp = pltpu.async_copy(preds_hbm.at[pl.ds(off, CH)], pb, ps)
            ct = pltpu.async_copy(targets_hbm.at[pl.ds(off, CH)], tb, ts)
            return cp, ct

        pending = start(0)
        pltpu.sync_copy(thr_hbm, thrv)

        zeros16 = jnp.zeros((LANES,), jnp.float32)

        @plsc.parallel_loop(0, ACCN, step=LANES, unroll=8)
        def _zero(i):
            acc[pl.ds(i, LANES)] = zeros16

        ones16 = jnp.full((LANES,), 1.0, jnp.float32)
        lane_iota = lax.iota(jnp.int32, 16)
        magic = jnp.float32(12582912.0)
        magic_bits = jnp.full((LANES,), 1262485504, jnp.int32)

        for c in range(NCHUNK):
            cur = pending
            if c + 1 < NCHUNK:
                pending = start(c + 1)
            cur[0].wait()
            cur[1].wait()
            pb, tb = bufs[c % 2][0], bufs[c % 2][1]

            @plsc.parallel_loop(0, CH, step=LANES, unroll=16)
            def _body(i):
                s = pl.ds(i, LANES)
                p = pb[s]
                v = p * 199.0 + magic
                g = plsc.bitcast(v, jnp.int32) - magic_bits
                gidx = (g << 4) + lane_iota
                tv = plsc.load_gather(thrv, [gidx])
                t = tb[s]
                idx = (gidx
                       + jnp.where(tv < p, 16, 0)
                       + jnp.where(t == 1.0, 4096, 0))
                plsc.addupdate_scatter(acc, [idx], ones16)

        pltpu.sync_copy(acc, out_hbm.at[wid])

    return k(preds, targets, thr)


def _tc_auc(partials):
    def body(h_ref, o_ref):
        h = h_ref[...]
        s = jnp.sum(h, axis=0, keepdims=True)
        h_nt = s[:, 0:256]
        h_t = s[:, 256:512]
        r = lax.broadcasted_iota(jnp.int32, (256, 256), 0)
        cc = lax.broadcasted_iota(jnp.int32, (256, 256), 1)
        m_cum = (r <= cc).astype(jnp.float32)
        m_rev = (jnp.maximum(r - 1, 0) >= cc).astype(jnp.float32)
        dot = functools.partial(lax.dot, precision=lax.Precision.HIGHEST)
        tn = dot(h_nt, m_cum)
        fn = dot(h_t, m_cum)
        fp = dot(h_nt, m_rev)
        tp = dot(h_t, m_rev)
        x = fp / (fp + tn + EPS)
        y = (tp + EPS) / (tp + fn + EPS)
        shift = (r == cc + 1).astype(jnp.float32)
        xs = dot(x, shift)
        ys = dot(y, shift)
        j = lax.broadcasted_iota(jnp.int32, (1, 256), 1)
        terms = jnp.where(j <= NT - 2, (x - xs) * (y + ys) * 0.5, 0.0)
        o_ref[...] = jnp.sum(terms, axis=1, keepdims=True)

    return pl.pallas_call(
        body,
        out_shape=jax.ShapeDtypeStruct((1, 1), jnp.float32),
    )(partials)


def kernel(preds, targets):
    p = preds.reshape(-1)
    t = targets.reshape(-1)
    hist = _sc_hist(p, t, _thr_table())
    parts = hist.reshape(NW, BINS, LANES).transpose(0, 2, 1).reshape(NW * LANES, BINS)
    roc = _tc_auc(parts)
    return roc.reshape(())
